# Initial kernel scaffold; baseline (speedup 1.0000x reference)
import jax
import jax.numpy as jnp
from jax.experimental import pallas as pl


def kernel(x, edge_index, W1l, W1r, a1, b1, W2l, W2r, a2, b2):
    raise NotImplementedError("write your pallas kernel here")


# speedup vs baseline: 7.5954x; 7.5954x over previous
import jax
import jax.numpy as jnp
from jax import lax
from jax.experimental import pallas as pl
from jax.experimental.pallas import tpu as pltpu
from jax.experimental.pallas import tpu_sc as plsc

N_NODES = 10000
N_EDGES = 320000
D = 128
HEADS = 4

NPAD = 10240
NC, NS, LANES = 2, 16, 16
NTILES = NC * NS
E_FULL = N_EDGES + N_NODES
EB = 128
BATCHES = 162
EPT = EB * BATCHES
EPAD = EPT * NS
HALF = NPAD // NC
ACC_ROWS = HALF + EB
ROWS_PER_TILE = ACC_ROWS // NS


NDG = NPAD // D
DG_SLAB = 8
DG_TILES = NDG // DG_SLAB


def _sc_att_body(xl_hbm, xr_hbm, att_hbm, src_hbm, dst_hbm,
                 feat_hbm, den_hbm,
                 src_v, dst_v, dst_pad, den_idx, loc_idx, xl_rows, xr_rows,
                 out_rows, den_rows, att_v, shared, den_acc, sem):
    c = lax.axis_index("c")
    s = lax.axis_index("s")
    lane = lax.iota(jnp.int32, LANES)

    pltpu.sync_copy(att_hbm, att_v)
    dst_pad[pl.ds(EB, LANES)] = jnp.zeros((LANES,), jnp.int32)
    att_c = [att_v[pl.ds(k * LANES, LANES)] for k in range(D // LANES)]
    lanefull = [lane + (k * LANES) for k in range(D // LANES)]

    def _zero_rows(r, carry):
        for kk in range(D // LANES):
            out_rows[r, pl.ds(kk * LANES, LANES)] = jnp.zeros((LANES,), jnp.float32)
        return carry

    lax.fori_loop(0, EB, _zero_rows, 0)

    if True:
        for off, nr in ((0, EB), (EB, EB), (2 * EB, ROWS_PER_TILE - 2 * EB)):
            pltpu.sync_copy(
                out_rows.at[pl.ds(0, nr)],
                shared.at[pl.ds(s * ROWS_PER_TILE + off, nr)])
        @pl.when(s < DG_TILES)
        def _zero_den():
            pltpu.sync_copy(out_rows.at[pl.ds(0, DG_SLAB)],
                            den_acc.at[pl.ds(s * DG_SLAB, DG_SLAB)])

        plsc.subcore_barrier()

        def _batch(b, carry):
            base = s * EPT + b * EB
            pltpu.sync_copy(src_hbm.at[pl.ds(base, EB)], src_v)
            pltpu.sync_copy(dst_hbm.at[pl.ds(base, EB)], dst_v)
            pltpu.sync_copy(dst_hbm.at[pl.ds(base, EB)], dst_pad.at[pl.ds(0, EB)])
            cp1 = pltpu.async_copy(xl_hbm.at[src_v], xl_rows, sem)
            cp2 = pltpu.async_copy(xr_hbm.at[dst_v], xr_rows, sem)
            for g in range(EB // LANES):
                dvg = dst_v[pl.ds(g * LANES, LANES)]
                den_idx[pl.ds(g * LANES, LANES)] = lax.shift_right_logical(dvg, 7)
                loc = dvg - c * HALF
                ok = (loc >= 0) & (loc < HALF)
                loc_idx[pl.ds(g * LANES, LANES)] = jnp.where(
                    ok, loc, jnp.full((LANES,), HALF, jnp.int32))
            cp1.wait()
            cp2.wait()

            def _edge(e, ecarry):
                xs = []
                acc = jnp.zeros((LANES,), jnp.float32)
                for k in range(D // LANES):
                    xlk = xl_rows[e, pl.ds(k * LANES, LANES)]
                    xs.append(xlk)
                    t = xlk + xr_rows[e, pl.ds(k * LANES, LANES)]
                    t = jnp.where(t >= 0.0, t, 0.2 * t)
                    acc = acc + t * att_c[k]
                for step in (8, 4, 2, 1):
                    acc = acc + acc.at[lane ^ step].get(mode="promise_in_bounds")
                pvec = jnp.exp(acc)
                dv = dst_pad[pl.ds(e, LANES)]
                dm = jnp.full((LANES,), dv[0] & (D - 1), jnp.int32)
                for k in range(D // LANES):
                    out_rows[e, pl.ds(k * LANES, LANES)] = xs[k] * pvec
                    den_rows[e, pl.ds(k * LANES, LANES)] = jnp.where(
                        lanefull[k] == dm, pvec, jnp.zeros((LANES,), jnp.float32))
                return ecarry

            lax.fori_loop(0, EB, _edge, 0)
            pltpu.sync_copy(out_rows, shared.at[loc_idx], add=True)
            pltpu.sync_copy(den_rows, den_acc.at[den_idx], add=True)
            return carry

        lax.fori_loop(0, BATCHES, _batch, 0)
        plsc.subcore_barrier()

        pltpu.sync_copy(shared.at[pl.ds(s * ROWS_PER_TILE, ROWS_PER_TILE)],
                        feat_hbm.at[c, pl.ds(s * ROWS_PER_TILE, ROWS_PER_TILE)])
        @pl.when(s < DG_TILES)
        def _dump_den():
            pltpu.sync_copy(den_acc.at[pl.ds(s * DG_SLAB, DG_SLAB)],
                            den_hbm.at[c, pl.ds(s * DG_SLAB, DG_SLAB)])


@jax.jit
def _sc_att(xl, xr, att, src, dst):
    mesh = plsc.VectorSubcoreMesh(core_axis_name="c", subcore_axis_name="s")
    return pl.kernel(
        _sc_att_body,
        out_type=(
            jax.ShapeDtypeStruct((NC, ACC_ROWS, D), jnp.float32),
            jax.ShapeDtypeStruct((NC, NDG, D), jnp.float32),
        ),
        mesh=mesh,
        scratch_types=[
            pltpu.VMEM((EB,), jnp.int32),
            pltpu.VMEM((EB,), jnp.int32),
            pltpu.VMEM((EB + LANES,), jnp.int32),
            pltpu.VMEM((EB,), jnp.int32),
            pltpu.VMEM((EB,), jnp.int32),
            pltpu.VMEM((EB, D), jnp.float32),
            pltpu.VMEM((EB, D), jnp.float32),
            pltpu.VMEM((EB, D), jnp.float32),
            pltpu.VMEM((EB, D), jnp.float32),
            pltpu.VMEM((D,), jnp.float32),
            pltpu.VMEM_SHARED((ACC_ROWS, D), jnp.float32),
            pltpu.VMEM_SHARED((NDG, D), jnp.float32),
            pltpu.SemaphoreType.DMA,
        ],
    )(xl, xr, att, src, dst)


_RT = 512
_NRT = NPAD // _RT


def _mm1_body(x_ref, w_ref, o_ref):
    o_ref[0] = jnp.dot(x_ref[...], w_ref[...], preferred_element_type=jnp.float32)


@jax.jit
def _mm1(xp, wcat):
    return pl.pallas_call(
        _mm1_body,
        grid=(2 * HEADS, _NRT),
        in_specs=[
            pl.BlockSpec((_RT, D), lambda j, i: (i, 0)),
            pl.BlockSpec((D, D), lambda j, i: (0, j)),
        ],
        out_specs=pl.BlockSpec((1, _RT, D), lambda j, i: (j, i, 0)),
        out_shape=jax.ShapeDtypeStruct((2 * HEADS, NPAD, D), jnp.float32),
    )(xp, wcat)


_HRT = HALF // _RT


def _norm_head(feat_ref, den_ref):
    num = feat_ref[0]
    den = den_ref[0][:, None]
    return num / (den + 1e-16)


def _mid_body(f0, f1, f2, f3, d0, d1, d2, d3, b1_ref, w_ref, o_ref):
    hs = [_norm_head(f, d) for f, d in ((f0, d0), (f1, d1), (f2, d2), (f3, d3))]
    h = jnp.concatenate(hs, axis=1) + b1_ref[0]
    h = jnp.where(h > 0.0, h, jnp.exp(jnp.minimum(h, 0.0)) - 1.0)
    o_ref[0] = jnp.dot(h, w_ref[...], preferred_element_type=jnp.float32)


@jax.jit
def _mid(p0, p1, p2, p3, b1r, wcat2):
    fspec = pl.BlockSpec((1, _RT, D), lambda j, i: (i // _HRT, i % _HRT, 0))
    dspec = pl.BlockSpec((NC, _RT), lambda j, i: (0, i))
    return pl.pallas_call(
        _mid_body,
        grid=(2, _NRT),
        in_specs=[fspec, fspec, fspec, fspec, dspec, dspec, dspec, dspec,
                  pl.BlockSpec((1, HEADS * D), lambda j, i: (0, 0)),
                  pl.BlockSpec((HEADS * D, D), lambda j, i: (0, j))],
        out_specs=pl.BlockSpec((1, _RT, D), lambda j, i: (j, i, 0)),
        out_shape=jax.ShapeDtypeStruct((2, NPAD, D), jnp.float32),
    )(p0[0], p1[0], p2[0], p3[0],
      p0[1].reshape(NC, NPAD), p1[1].reshape(NC, NPAD),
      p2[1].reshape(NC, NPAD), p3[1].reshape(NC, NPAD), b1r, wcat2)


def _fin_body(f_ref, d_ref, b2_ref, o_ref):
    o_ref[...] = _norm_head(f_ref, d_ref) + b2_ref[0]


@jax.jit
def _fin(q, b2r):
    return pl.pallas_call(
        _fin_body,
        grid=(_NRT,),
        in_specs=[
            pl.BlockSpec((1, _RT, D), lambda i: (i // _HRT, i % _HRT, 0)),
            pl.BlockSpec((NC, _RT), lambda i: (0, i)),
            pl.BlockSpec((1, D), lambda i: (0, 0)),
        ],
        out_specs=pl.BlockSpec((_RT, D), lambda i: (i, 0)),
        out_shape=jax.ShapeDtypeStruct((NPAD, D), jnp.float32),
    )(q[0], q[1].reshape(NC, NPAD), b2r)


def kernel(x, edge_index, W1l, W1r, a1, b1, W2l, W2r, a2, b2):
    xp = jnp.zeros((NPAD, D), jnp.float32).at[:N_NODES].set(x)
    loop = jnp.arange(N_NODES, dtype=jnp.int32)
    pad = jnp.full((EPAD - E_FULL,), N_NODES, dtype=jnp.int32)
    src = jnp.concatenate([edge_index[0].astype(jnp.int32), loop, pad])
    dst = jnp.concatenate([edge_index[1].astype(jnp.int32), loop, pad])

    wcat1 = jnp.concatenate([W1l, W1r], axis=1)
    y1 = _mm1(xp, wcat1)

    parts = [
        _sc_att(y1[h], y1[HEADS + h], a1[h], src, dst) for h in range(HEADS)
    ]

    wcat2 = jnp.concatenate([W2l, W2r], axis=1)
    y2 = _mid(parts[0], parts[1], parts[2], parts[3],
              b1.reshape(1, HEADS * D), wcat2)

    q = _sc_att(y2[0], y2[1], a2[0], src, dst)
    out = _fin(q, b2.reshape(1, D))
    return out[:N_NODES]
